# baseline (speedup 1.0000x reference)
import functools

import jax
import jax.numpy as jnp
from jax import lax
from jax.experimental import pallas as pl
from jax.experimental.pallas import tpu as pltpu
from jax.experimental.pallas import tpu_sc as plsc

_N = 32 * 3 * 512 * 512
_NC, _NS, _L = 2, 16, 16
_NW = _NC * _NS
_PER_W = _N // _NW
_CHUNK = 32768
_NCHUNK = _PER_W // _CHUNK
_VECS = _CHUNK // _L

_ROWS = _N // 1024
_BR = 2048
_G = _ROWS // _BR


_sc_mesh = plsc.VectorSubcoreMesh(core_axis_name="c", subcore_axis_name="s")


@functools.partial(
    pl.kernel,
    out_type=jax.ShapeDtypeStruct((_NW, _L, 256), jnp.int32),
    mesh=_sc_mesh,
    compiler_params=pltpu.CompilerParams(needs_layout_passes=False),
    scratch_types=[
        pltpu.VMEM((_CHUNK,), jnp.float32),
        pltpu.VMEM((_CHUNK,), jnp.float32),
        pltpu.VMEM((256 * _L,), jnp.int32),
        pltpu.VMEM((_L, 256), jnp.int32),
        pltpu.SemaphoreType.DMA,
        pltpu.SemaphoreType.DMA,
    ],
)
def _sc_hist(x_hbm, out_hbm, buf0, buf1, hist2, histt, sem0, sem1):
    wid = lax.axis_index("s") * _NC + lax.axis_index("c")
    base = wid * _PER_W

    zero = jnp.zeros((_L,), jnp.int32)

    @plsc.parallel_loop(0, 256 * _L, step=_L)
    def _zero_body(r):
        hist2[pl.ds(r, _L)] = zero

    bufs = (buf0, buf1)
    sems = (sem0, sem1)
    copies = [None, None]
    lane = lax.broadcasted_iota(jnp.int32, (_L,), 0)
    one = jnp.ones((_L,), jnp.int32)

    copies[0] = pltpu.async_copy(x_hbm.at[pl.ds(base, _CHUNK)], buf0, sem0)
    for c in range(_NCHUNK):
        if c + 1 < _NCHUNK:
            nxt = (c + 1) % 2
            copies[nxt] = pltpu.async_copy(
                x_hbm.at[pl.ds(base + (c + 1) * _CHUNK, _CHUNK)],
                bufs[nxt],
                sems[nxt],
            )
        copies[c % 2].wait()
        cur = bufs[c % 2]

        @plsc.parallel_loop(0, _CHUNK, step=_L, unroll=8)
        def _chunk_body(i):
            x = cur[pl.ds(i, _L)]
            idx = (x * 256.0).astype(jnp.int32)
            slot = jnp.left_shift(idx, 4) | lane
            plsc.addupdate_scatter(hist2, [slot], one)

    @plsc.parallel_loop(0, _L * 256, step=_L, unroll=4)
    def _tr_body(j):
        lane_out = jnp.right_shift(j, 8)
        bin_base = j & 255
        src = jnp.left_shift(bin_base + lane, 4) | lane_out
        histt[lane_out, pl.ds(bin_base, _L)] = plsc.load_gather(hist2, [src])

    pltpu.sync_copy(histt, out_hbm.at[wid])


def _tc_mse_body(o_ref, i_ref, sq_ref, acc):
    step = pl.program_id(0)

    @pl.when(step == 0)
    def _init():
        acc[0, 0] = 0.0

    d = o_ref[...] - i_ref[...]
    acc[0, 0] += jnp.sum(d * d)

    @pl.when(step == _G - 1)
    def _fini():
        sq_ref[0, 0] = acc[0, 0]


def _tc_combine_body(hist_ref, sq_ref, loss_ref, bpp_ref, dist_ref):
    counts = jnp.sum(hist_ref[...].astype(jnp.float32), axis=0)
    total = jnp.sum(counts)
    p = counts / total
    p = jnp.clip(p, 1e-12, 1.0)
    ent = -jnp.sum(p * jnp.log2(p))
    bpp = ent / 32.0
    dist = sq_ref[0, 0] / float(_N)
    bpp_ref[0, 0] = bpp
    dist_ref[0, 0] = dist
    loss_ref[0, 0] = bpp + dist


@jax.jit
def kernel(outputs, inputs):
    flat_o = outputs.reshape(_N)
    hist = jnp.ones((_NW, _L, 256), jnp.int32)
    o2 = flat_o.reshape(_ROWS, 1024)
    i2 = inputs.reshape(_ROWS, 1024)
    sq = pl.pallas_call(
        _tc_mse_body,
        grid=(_G,),
        in_specs=[
            pl.BlockSpec((_BR, 1024), lambda i: (i, 0)),
            pl.BlockSpec((_BR, 1024), lambda i: (i, 0)),
        ],
        out_specs=pl.BlockSpec(memory_space=pltpu.SMEM),
        out_shape=jax.ShapeDtypeStruct((1, 1), jnp.float32),
        scratch_shapes=[pltpu.SMEM((1, 1), jnp.float32)],
    )(o2, i2)
    loss, bpp, dist = pl.pallas_call(
        _tc_combine_body,
        in_specs=[
            pl.BlockSpec((_NW * _L, 256), lambda: (0, 0)),
            pl.BlockSpec(memory_space=pltpu.SMEM),
        ],
        out_specs=[
            pl.BlockSpec(memory_space=pltpu.SMEM),
            pl.BlockSpec(memory_space=pltpu.SMEM),
            pl.BlockSpec(memory_space=pltpu.SMEM),
        ],
        out_shape=[jax.ShapeDtypeStruct((1, 1), jnp.float32)] * 3,
    )(hist.reshape(_NW * _L, 256), sq)
    return loss[0, 0], bpp[0, 0], dist[0, 0]


# speedup vs baseline: 1.5188x; 1.5188x over previous
import functools

import jax
import jax.numpy as jnp
from jax import lax
from jax.experimental import pallas as pl
from jax.experimental.pallas import tpu as pltpu
from jax.experimental.pallas import tpu_sc as plsc

_N = 32 * 3 * 512 * 512
_NC, _NS, _L = 2, 16, 16
_NW = _NC * _NS
_PER_W = _N // _NW
_CHUNK = 32768
_NCHUNK = _PER_W // _CHUNK
_VECS = _CHUNK // _L

_ROWS = _N // 1024
_BR = 2048
_G = _ROWS // _BR


_sc_mesh = plsc.VectorSubcoreMesh(core_axis_name="c", subcore_axis_name="s")


@functools.partial(
    pl.kernel,
    out_type=jax.ShapeDtypeStruct((_NW, _L, 256), jnp.int32),
    mesh=_sc_mesh,
    compiler_params=pltpu.CompilerParams(needs_layout_passes=False),
    scratch_types=[
        pltpu.VMEM((_CHUNK,), jnp.float32),
        pltpu.VMEM((_CHUNK,), jnp.float32),
        pltpu.VMEM((256 * _L,), jnp.int32),
        pltpu.VMEM((_L, 256), jnp.int32),
        pltpu.SemaphoreType.DMA,
        pltpu.SemaphoreType.DMA,
    ],
)
def _sc_hist(x_hbm, out_hbm, buf0, buf1, hist2, histt, sem0, sem1):
    wid = lax.axis_index("s") * _NC + lax.axis_index("c")
    base = wid * _PER_W

    zero = jnp.zeros((_L,), jnp.int32)

    @plsc.parallel_loop(0, 256 * _L, step=_L)
    def _zero_body(r):
        hist2[pl.ds(r, _L)] = zero

    bufs = (buf0, buf1)
    sems = (sem0, sem1)
    copies = [None, None]
    lane = lax.broadcasted_iota(jnp.int32, (_L,), 0)
    one = jnp.ones((_L,), jnp.int32)

    copies[0] = pltpu.async_copy(x_hbm.at[pl.ds(base, _CHUNK)], buf0, sem0)
    for c in range(_NCHUNK):
        if c + 1 < _NCHUNK:
            nxt = (c + 1) % 2
            copies[nxt] = pltpu.async_copy(
                x_hbm.at[pl.ds(base + (c + 1) * _CHUNK, _CHUNK)],
                bufs[nxt],
                sems[nxt],
            )
        copies[c % 2].wait()
        cur = bufs[c % 2]

        @plsc.parallel_loop(0, _CHUNK, step=_L, unroll=8)
        def _chunk_body(i):
            x = cur[pl.ds(i, _L)]
            idx = (x * 256.0).astype(jnp.int32)
            slot = jnp.left_shift(idx, 4) | lane
            plsc.addupdate_scatter(hist2, [slot], one)

    @plsc.parallel_loop(0, _L * 256, step=_L, unroll=4)
    def _tr_body(j):
        lane_out = jnp.right_shift(j, 8)
        bin_base = j & 255
        src = jnp.left_shift(bin_base + lane, 4) | lane_out
        histt[lane_out, pl.ds(bin_base, _L)] = plsc.load_gather(hist2, [src])

    pltpu.sync_copy(histt, out_hbm.at[wid])


def _tc_mse_body(o_ref, i_ref, sq_ref, acc):
    step = pl.program_id(0)

    @pl.when(step == 0)
    def _init():
        acc[0, 0] = 0.0

    d = o_ref[...] - i_ref[...]
    acc[0, 0] += jnp.sum(d * d)

    @pl.when(step == 31)
    def _fini():
        sq_ref[0, 0] = acc[0, 0]


def _tc_combine_body(hist_ref, sq_ref, loss_ref, bpp_ref, dist_ref):
    counts = jnp.sum(hist_ref[...].astype(jnp.float32), axis=0)
    total = jnp.sum(counts)
    p = counts / total
    p = jnp.clip(p, 1e-12, 1.0)
    ent = -jnp.sum(p * jnp.log2(p))
    bpp = ent / 32.0
    dist = sq_ref[0, 0] / float(_N)
    bpp_ref[0, 0] = bpp
    dist_ref[0, 0] = dist
    loss_ref[0, 0] = bpp + dist


@jax.jit
def kernel(outputs, inputs):
    hist = _sc_hist(outputs.reshape(_N))
    sq = pl.pallas_call(
        _tc_mse_body,
        grid=(32,),
        in_specs=[
            pl.BlockSpec((1, 3, 512, 512), lambda i: (i, 0, 0, 0)),
            pl.BlockSpec((1, 3, 512, 512), lambda i: (i, 0, 0, 0)),
        ],
        out_specs=pl.BlockSpec(memory_space=pltpu.SMEM),
        out_shape=jax.ShapeDtypeStruct((1, 1), jnp.float32),
        scratch_shapes=[pltpu.SMEM((1, 1), jnp.float32)],
    )(outputs, inputs)
    loss, bpp, dist = pl.pallas_call(
        _tc_combine_body,
        in_specs=[
            pl.BlockSpec((_NW * _L, 256), lambda: (0, 0)),
            pl.BlockSpec(memory_space=pltpu.SMEM),
        ],
        out_specs=[
            pl.BlockSpec(memory_space=pltpu.SMEM),
            pl.BlockSpec(memory_space=pltpu.SMEM),
            pl.BlockSpec(memory_space=pltpu.SMEM),
        ],
        out_shape=[jax.ShapeDtypeStruct((1, 1), jnp.float32)] * 3,
    )(hist.reshape(_NW * _L, 256), sq)
    return loss[0, 0], bpp[0, 0], dist[0, 0]


# speedup vs baseline: 2.4715x; 1.6273x over previous
import functools

import jax
import jax.numpy as jnp
from jax import lax
from jax.experimental import pallas as pl
from jax.experimental.pallas import tpu as pltpu
from jax.experimental.pallas import tpu_sc as plsc

_N = 32 * 3 * 512 * 512
_NC, _NS, _L = 2, 16, 16
_NW = _NC * _NS
_PER_W = _N // _NW
_CHUNK = 32768
_NCHUNK = _PER_W // _CHUNK
_VECS = _CHUNK // _L

_ROWS = _N // 1024
_BR = 2048
_G = _ROWS // _BR


_sc_mesh = plsc.VectorSubcoreMesh(core_axis_name="c", subcore_axis_name="s")


@functools.partial(
    pl.kernel,
    out_type=jax.ShapeDtypeStruct((_NW, _L, 256), jnp.int32),
    mesh=_sc_mesh,
    compiler_params=pltpu.CompilerParams(
        needs_layout_passes=False, use_tc_tiling_on_sc=True
    ),
    scratch_types=[
        pltpu.VMEM((64, 512), jnp.float32),
        pltpu.VMEM((64, 512), jnp.float32),
        pltpu.VMEM((256 * _L,), jnp.int32),
        pltpu.VMEM((_L, 256), jnp.int32),
        pltpu.SemaphoreType.DMA,
        pltpu.SemaphoreType.DMA,
    ],
)
def _sc_hist(x_hbm, out_hbm, buf0, buf1, hist2, histt, sem0, sem1):
    wid = lax.axis_index("s") * _NC + lax.axis_index("c")

    zero = jnp.zeros((_L,), jnp.int32)

    @plsc.parallel_loop(0, 256 * _L, step=_L)
    def _zero_body(r):
        hist2[pl.ds(r, _L)] = zero

    bufs = (buf0, buf1)
    sems = (sem0, sem1)
    copies = [None, None]
    lane = lax.broadcasted_iota(jnp.int32, (_L,), 0)
    one = jnp.ones((_L,), jnp.int32)

    copies[0] = pltpu.async_copy(
        x_hbm.at[wid, 0, pl.ds(0, 64), :], buf0, sem0
    )
    for c in range(_NCHUNK):
        if c + 1 < _NCHUNK:
            nxt = (c + 1) % 2
            ch, r0 = divmod(c + 1, 8)
            copies[nxt] = pltpu.async_copy(
                x_hbm.at[wid, ch, pl.ds(r0 * 64, 64), :],
                bufs[nxt],
                sems[nxt],
            )
        copies[c % 2].wait()
        cur = bufs[c % 2]

        @plsc.parallel_loop(0, _CHUNK, step=_L, unroll=8)
        def _chunk_body(i):
            x = cur[jnp.right_shift(i, 9), pl.ds(i & 511, _L)]
            idx = (x * 256.0).astype(jnp.int32)
            slot = jnp.left_shift(idx, 4) | lane
            plsc.addupdate_scatter(hist2, [slot], one)

    @plsc.parallel_loop(0, _L * 256, step=_L, unroll=4)
    def _tr_body(j):
        lane_out = jnp.right_shift(j, 8)
        bin_base = j & 255
        src = jnp.left_shift(bin_base + lane, 4) | lane_out
        histt[lane_out, pl.ds(bin_base, _L)] = plsc.load_gather(hist2, [src])

    pltpu.sync_copy(histt, out_hbm.at[wid])


def _tc_mse_body(o_ref, i_ref, sq_ref, acc):
    step = pl.program_id(0)

    @pl.when(step == 0)
    def _init():
        acc[0, 0] = 0.0

    d = o_ref[...] - i_ref[...]
    acc[0, 0] += jnp.sum(d * d)

    @pl.when(step == 31)
    def _fini():
        sq_ref[0, 0] = acc[0, 0]


def _tc_combine_body(hist_ref, sq_ref, loss_ref, bpp_ref, dist_ref):
    counts = jnp.sum(hist_ref[...].astype(jnp.float32), axis=0)
    total = jnp.sum(counts)
    p = counts / total
    p = jnp.clip(p, 1e-12, 1.0)
    ent = -jnp.sum(p * jnp.log2(p))
    bpp = ent / 32.0
    dist = sq_ref[0, 0] / float(_N)
    bpp_ref[0, 0] = bpp
    dist_ref[0, 0] = dist
    loss_ref[0, 0] = bpp + dist


@jax.jit
def kernel(outputs, inputs):
    hist = _sc_hist(outputs)
    sq = pl.pallas_call(
        _tc_mse_body,
        grid=(32,),
        in_specs=[
            pl.BlockSpec((1, 3, 512, 512), lambda i: (i, 0, 0, 0)),
            pl.BlockSpec((1, 3, 512, 512), lambda i: (i, 0, 0, 0)),
        ],
        out_specs=pl.BlockSpec(memory_space=pltpu.SMEM),
        out_shape=jax.ShapeDtypeStruct((1, 1), jnp.float32),
        scratch_shapes=[pltpu.SMEM((1, 1), jnp.float32)],
    )(outputs, inputs)
    loss, bpp, dist = pl.pallas_call(
        _tc_combine_body,
        in_specs=[
            pl.BlockSpec((_NW * _L, 256), lambda: (0, 0)),
            pl.BlockSpec(memory_space=pltpu.SMEM),
        ],
        out_specs=[
            pl.BlockSpec(memory_space=pltpu.SMEM),
            pl.BlockSpec(memory_space=pltpu.SMEM),
            pl.BlockSpec(memory_space=pltpu.SMEM),
        ],
        out_shape=[jax.ShapeDtypeStruct((1, 1), jnp.float32)] * 3,
    )(hist.reshape(_NW * _L, 256), sq)
    return loss[0, 0], bpp[0, 0], dist[0, 0]
